# baseline (speedup 1.0000x reference)
import functools

import jax
import jax.numpy as jnp
from jax import lax
from jax.experimental import pallas as pl
from jax.experimental.pallas import tpu as pltpu
from jax.experimental.pallas import tpu_sc as plsc

N = 10000
F = 128
NCORES = 2
NSUB = 16
NTILES = NCORES * NSUB
CHUNK = 128
DEG_PAD = 10240
DSLICE = DEG_PAD // NSUB
ROWS_PER_TILE = DEG_PAD // NSUB
ZBLK = 128
NSELF = -(-N // CHUNK)
SELF_LAST = (NSELF - 2) * CHUNK


def _rsqrt16(d):
    i = plsc.bitcast(d, jnp.int32)
    i = jnp.full((16,), 0x5F3759DF, jnp.int32) - lax.shift_right_logical(i, 1)
    y = plsc.bitcast(i, jnp.float32)
    half_d = d * 0.5
    for _ in range(3):
        y = y * (1.5 - half_d * y * y)
    return y


def _sc_deg_kernel(nreal, nsteps, sidx_hbm, ew_hbm, zeros_hbm, out_hbm,
                   sbuf0, sbuf1, ewbuf0, ewbuf1, scidx0, scidx1,
                   scdat0, scdat1, bounce, msem0, msem1, ssem0, ssem1,
                   deg_spmem):
    c = lax.axis_index("c")
    s = lax.axis_index("s")
    t = c * NSUB + s

    @pl.when(s == 0)
    def _zero():
        pltpu.sync_copy(zeros_hbm, bounce)
        pltpu.sync_copy(bounce, deg_spmem)

    plsc.subcore_barrier()

    sbufs = (sbuf0, sbuf1)
    ewbufs = (ewbuf0, ewbuf1)
    scidxs = (scidx0, scidx1)
    scdats = (scdat0, scdat1)
    msems = (msem0, msem1)
    ssems = (ssem0, ssem1)

    def _valid(m):
        return m * NTILES + t < nreal

    def _issue_meta(m, slot):
        base = (m * NTILES + t) * CHUNK
        pltpu.async_copy(sidx_hbm.at[pl.ds(base, CHUNK)], sbufs[slot], msems[slot])
        pltpu.async_copy(ew_hbm.at[pl.ds(base, CHUNK)], ewbufs[slot], msems[slot])

    def _wait_meta(m, slot):
        base = (m * NTILES + t) * CHUNK
        pltpu.make_async_copy(sidx_hbm.at[pl.ds(base, CHUNK)], sbufs[slot], msems[slot]).wait()
        pltpu.make_async_copy(ew_hbm.at[pl.ds(base, CHUNK)], ewbufs[slot], msems[slot]).wait()

    def _drain_scatter(slot):
        pltpu.make_async_copy(scdats[slot], deg_spmem.at[scidxs[slot]], ssems[slot]).wait()

    pltpu.sync_copy(sidx_hbm.at[pl.ds(t * CHUNK, CHUNK)], sbuf0)
    pltpu.sync_copy(ew_hbm.at[pl.ds(t * CHUNK, CHUNK)], ewbuf0)
    _issue_meta(1, 1)

    def _step(m, cur, nxt):
        @pl.when(jnp.logical_and(m >= 2, _valid(m)))
        def _dr():
            _drain_scatter(cur)
        @pl.when(_valid(m + 1))
        def _w():
            _wait_meta(m + 1, nxt)
        @pl.when(_valid(m))
        def _sc():
            for g in range(CHUNK // 16):
                sl = pl.ds(g * 16, 16)
                scidxs[cur][sl] = sbufs[cur][sl]
                scdats[cur][sl] = ewbufs[cur][sl]
            pltpu.async_copy(scdats[cur], deg_spmem.at[scidxs[cur]], ssems[cur], add=True)
        @pl.when(_valid(m + 2))
        def _i():
            _issue_meta(m + 2, cur)

    def chunk(j, carry):
        _step(2 * j, 0, 1)
        _step(2 * j + 1, 1, 0)
        return carry

    lax.fori_loop(0, nsteps // 2, chunk, 0)
    _drain_scatter(0)
    _drain_scatter(1)
    plsc.subcore_barrier()

    @pl.when(s == 0)
    def _out():
        pltpu.sync_copy(deg_spmem, bounce)
        pltpu.sync_copy(bounce, out_hbm.at[pl.ds(c * DEG_PAD, DEG_PAD)])


def _sc_agg_kernel(nreal, nsteps, rows_hbm, sidx_hbm, ew_hbm, deg_hbm, x_hbm,
                   zeros2_hbm, out_hbm,
                   rbuf0, rbuf1, sbuf0, sbuf1, ewbuf0, ewbuf1, nbuf,
                   scidx0, scidx1, db0, db1, disbuf, xrows0, xrows1,
                   msem0, msem1, gsem0, gsem1, ssem0, ssem1,
                   dis_spmem, agg_spmem):
    c = lax.axis_index("c")
    s = lax.axis_index("s")
    t = c * NSUB + s
    lanes = lax.iota(jnp.int32, 16)

    dbase = s * DSLICE
    pltpu.sync_copy(deg_hbm.at[pl.ds(dbase, DSLICE)], db0)
    pltpu.sync_copy(deg_hbm.at[pl.ds(DEG_PAD + dbase, DSLICE)], db1)
    for g in range(DSLICE // 16):
        sl = pl.ds(g * 16, 16)
        db0[sl] = _rsqrt16(db0[sl] + db1[sl] + 1.0)
    pltpu.sync_copy(db0, dis_spmem.at[pl.ds(dbase, DSLICE)])

    pltpu.sync_copy(zeros2_hbm, xrows0)
    for k in range(ROWS_PER_TILE // ZBLK):
        pltpu.sync_copy(xrows0.at[pl.ds(0, ZBLK)],
                        agg_spmem.at[pl.ds(s * ROWS_PER_TILE + k * ZBLK, ZBLK)])
    plsc.subcore_barrier()
    pltpu.sync_copy(dis_spmem, disbuf)

    rbufs = (rbuf0, rbuf1)
    sbufs = (sbuf0, sbuf1)
    ewbufs = (ewbuf0, ewbuf1)
    scidxs = (scidx0, scidx1)
    xrows = (xrows0, xrows1)
    msems = (msem0, msem1)
    gsems = (gsem0, gsem1)
    ssems = (ssem0, ssem1)

    def _valid(m):
        return m * NTILES + t < nreal

    def _issue_meta(m, slot):
        base = (m * NTILES + t) * CHUNK
        pltpu.async_copy(rows_hbm.at[pl.ds(base, CHUNK)], rbufs[slot], msems[slot])
        pltpu.async_copy(sidx_hbm.at[pl.ds(base, CHUNK)], sbufs[slot], msems[slot])
        pltpu.async_copy(ew_hbm.at[pl.ds(base, CHUNK)], ewbufs[slot], msems[slot])

    def _wait_meta(m, slot):
        base = (m * NTILES + t) * CHUNK
        pltpu.make_async_copy(rows_hbm.at[pl.ds(base, CHUNK)], rbufs[slot], msems[slot]).wait()
        pltpu.make_async_copy(sidx_hbm.at[pl.ds(base, CHUNK)], sbufs[slot], msems[slot]).wait()
        pltpu.make_async_copy(ew_hbm.at[pl.ds(base, CHUNK)], ewbufs[slot], msems[slot]).wait()

    def _drain_scatter(slot):
        pltpu.make_async_copy(xrows[slot], agg_spmem.at[scidxs[slot]], ssems[slot]).wait()

    def _scale_rows(buf, nrows):
        @plsc.parallel_loop(0, nrows, unroll=8)
        def edge(e):
            ev = jnp.full((16,), 0, jnp.int32) + e
            ns = plsc.load_gather(nbuf, [ev])
            for l in range(F // 16):
                buf[e, pl.ds(l * 16, 16)] = buf[e, pl.ds(l * 16, 16)] * ns

    pltpu.sync_copy(rows_hbm.at[pl.ds(t * CHUNK, CHUNK)], rbuf0)
    pltpu.sync_copy(sidx_hbm.at[pl.ds(t * CHUNK, CHUNK)], sbuf0)
    pltpu.sync_copy(ew_hbm.at[pl.ds(t * CHUNK, CHUNK)], ewbuf0)
    pltpu.async_copy(x_hbm.at[rbuf0], xrows0, gsem0)
    _issue_meta(1, 1)

    def _step(m, cur, nxt):
        @pl.when(_valid(m + 1))
        def _wi():
            _wait_meta(m + 1, nxt)
            @pl.when(m >= 1)
            def _drs():
                _drain_scatter(nxt)
            pltpu.async_copy(x_hbm.at[rbufs[nxt]], xrows[nxt], gsems[nxt])
        @pl.when(_valid(m))
        def _body():
            pltpu.make_async_copy(x_hbm.at[rbufs[cur]], xrows[cur], gsems[cur]).wait()
            for g in range(CHUNK // 16):
                sl = pl.ds(g * 16, 16)
                r16 = rbufs[cur][sl]
                c16 = sbufs[cur][sl]
                e16 = ewbufs[cur][sl]
                nbuf[sl] = plsc.load_gather(disbuf, [r16]) * e16 * plsc.load_gather(disbuf, [c16])
                scidxs[cur][sl] = c16
            @pl.when(_valid(m + 2))
            def _im():
                _issue_meta(m + 2, cur)
            _scale_rows(xrows[cur], CHUNK)
            pltpu.async_copy(xrows[cur], agg_spmem.at[scidxs[cur]], ssems[cur], add=True)

    def chunk(j, carry):
        _step(2 * j, 0, 1)
        _step(2 * j + 1, 1, 0)
        return carry

    lax.fori_loop(0, nsteps // 2, chunk, 0)
    _drain_scatter(0)
    _drain_scatter(1)

    def _self_unit(u, carry):
        @pl.when(u * NTILES + t < NSELF)
        def _do():
            uu = u * NTILES + t
            nb = jnp.minimum(uu * CHUNK, N - CHUNK)
            thr = jnp.where(uu == NSELF - 1, SELF_LAST + CHUNK, 0)
            pltpu.sync_copy(x_hbm.at[pl.ds(nb, CHUNK)], xrows0)
            for g in range(CHUNK // 16):
                sl = pl.ds(g * 16, 16)
                r16 = lanes + (nb + g * 16)
                d16 = plsc.load_gather(disbuf, [r16])
                nbuf[sl] = jnp.where(r16 >= thr, d16 * d16, 0.0)
                scidx0[sl] = r16
            _scale_rows(xrows0, CHUNK)
            pltpu.sync_copy(xrows0, agg_spmem.at[scidx0], add=True)
        return carry

    lax.fori_loop(0, -(-NSELF // NTILES), _self_unit, 0)
    plsc.subcore_barrier()

    for k in range(ROWS_PER_TILE // ZBLK):
        base = s * ROWS_PER_TILE + k * ZBLK
        pltpu.sync_copy(agg_spmem.at[pl.ds(base, ZBLK)], xrows0.at[pl.ds(0, ZBLK)])
        pltpu.sync_copy(xrows0.at[pl.ds(0, ZBLK)], out_hbm.at[c, pl.ds(base, ZBLK)])


def _tc_mlp_kernel(agg_ref, w1_ref, b1_ref, wl1_ref, bl1_ref, wl2_ref,
                   bl2_ref, wl3_ref, bl3_ref, out_ref):
    a = agg_ref[0] + agg_ref[1]
    h = jnp.dot(a, w1_ref[...], preferred_element_type=jnp.float32) + b1_ref[...]
    h = jnp.maximum(h, 0.0)
    h = jnp.dot(h, wl1_ref[...], preferred_element_type=jnp.float32) + bl1_ref[...]
    h = jnp.maximum(h, 0.0)
    h = jnp.dot(h, wl2_ref[...], preferred_element_type=jnp.float32) + bl2_ref[...]
    h = jnp.maximum(h, 0.0)
    o = jnp.dot(h, wl3_ref[...], preferred_element_type=jnp.float32) + bl3_ref[...]
    m = jnp.max(o, axis=1, keepdims=True)
    ex = jnp.exp(o - m)
    out_ref[...] = ex / jnp.sum(ex, axis=1, keepdims=True)


def kernel(x, edge_index, edge_weight, W1, b1, Wl1, bl1, Wl2, bl2, Wl3, bl3):
    E = edge_weight.shape[0]
    assert E % CHUNK == 0
    nreal = E // CHUNK
    nsteps = -(-nreal // (2 * NTILES)) * 2

    ei_flat = edge_index.reshape(-1)
    rows_flat = ei_flat[:E]
    cols_flat = ei_flat[E:]
    zeros1 = jnp.zeros((DEG_PAD,), jnp.float32)
    zeros2 = jnp.zeros((CHUNK, F), jnp.float32)

    mesh = plsc.VectorSubcoreMesh(core_axis_name="c", subcore_axis_name="s")
    sc_params = pltpu.CompilerParams(needs_layout_passes=False)

    deg_parts = pl.kernel(
        functools.partial(_sc_deg_kernel, nreal, nsteps),
        mesh=mesh,
        out_type=jax.ShapeDtypeStruct((NCORES * DEG_PAD,), jnp.float32),
        scratch_types=[
            pltpu.VMEM((CHUNK,), jnp.int32),
            pltpu.VMEM((CHUNK,), jnp.int32),
            pltpu.VMEM((CHUNK,), jnp.float32),
            pltpu.VMEM((CHUNK,), jnp.float32),
            pltpu.VMEM((CHUNK,), jnp.int32),
            pltpu.VMEM((CHUNK,), jnp.int32),
            pltpu.VMEM((CHUNK,), jnp.float32),
            pltpu.VMEM((CHUNK,), jnp.float32),
            pltpu.VMEM((DEG_PAD,), jnp.float32),
            pltpu.SemaphoreType.DMA,
            pltpu.SemaphoreType.DMA,
            pltpu.SemaphoreType.DMA,
            pltpu.SemaphoreType.DMA,
            pltpu.VMEM_SHARED((DEG_PAD,), jnp.float32),
        ],
        compiler_params=sc_params,
    )(cols_flat, edge_weight, zeros1)

    agg = pl.kernel(
        functools.partial(_sc_agg_kernel, nreal, nsteps),
        mesh=mesh,
        out_type=jax.ShapeDtypeStruct((NCORES, DEG_PAD, F), jnp.float32),
        scratch_types=[
            pltpu.VMEM((CHUNK,), jnp.int32),
            pltpu.VMEM((CHUNK,), jnp.int32),
            pltpu.VMEM((CHUNK,), jnp.int32),
            pltpu.VMEM((CHUNK,), jnp.int32),
            pltpu.VMEM((CHUNK,), jnp.float32),
            pltpu.VMEM((CHUNK,), jnp.float32),
            pltpu.VMEM((CHUNK,), jnp.float32),
            pltpu.VMEM((CHUNK,), jnp.int32),
            pltpu.VMEM((CHUNK,), jnp.int32),
            pltpu.VMEM((DSLICE,), jnp.float32),
            pltpu.VMEM((DSLICE,), jnp.float32),
            pltpu.VMEM((DEG_PAD,), jnp.float32),
            pltpu.VMEM((CHUNK, F), jnp.float32),
            pltpu.VMEM((CHUNK, F), jnp.float32),
            pltpu.SemaphoreType.DMA,
            pltpu.SemaphoreType.DMA,
            pltpu.SemaphoreType.DMA,
            pltpu.SemaphoreType.DMA,
            pltpu.SemaphoreType.DMA,
            pltpu.SemaphoreType.DMA,
            pltpu.VMEM_SHARED((DEG_PAD,), jnp.float32),
            pltpu.VMEM_SHARED((DEG_PAD, F), jnp.float32),
        ],
        compiler_params=sc_params,
    )(rows_flat, cols_flat, edge_weight, deg_parts, x, zeros2)

    blk = 1000
    grid = (N // blk,)
    out = pl.pallas_call(
        _tc_mlp_kernel,
        grid=grid,
        in_specs=[
            pl.BlockSpec((NCORES, blk, F), lambda i: (0, i, 0)),
            pl.BlockSpec(W1.shape, lambda i: (0, 0)),
            pl.BlockSpec((1, b1.shape[0]), lambda i: (0, 0)),
            pl.BlockSpec(Wl1.shape, lambda i: (0, 0)),
            pl.BlockSpec((1, bl1.shape[0]), lambda i: (0, 0)),
            pl.BlockSpec(Wl2.shape, lambda i: (0, 0)),
            pl.BlockSpec((1, bl2.shape[0]), lambda i: (0, 0)),
            pl.BlockSpec(Wl3.shape, lambda i: (0, 0)),
            pl.BlockSpec((1, bl3.shape[0]), lambda i: (0, 0)),
        ],
        out_specs=pl.BlockSpec((blk, Wl3.shape[1]), lambda i: (i, 0)),
        out_shape=jax.ShapeDtypeStruct((N, Wl3.shape[1]), jnp.float32),
    )(agg, W1, b1.reshape(1, -1), Wl1, bl1.reshape(1, -1),
      Wl2, bl2.reshape(1, -1), Wl3, bl3.reshape(1, -1))
    return out
